# baseline (speedup 1.0000x reference)
import functools
import math

import jax
import jax.numpy as jnp
from jax import lax
from jax.experimental import pallas as pl
from jax.experimental.pallas import tpu as pltpu
from jax.experimental.pallas import tpu_sc as plsc

C0, C1 = 20000, 60000
D0, D1, D2 = 1024, 256, 64
D2P = 256
OUT_DIM = 1024
SCALE = math.sqrt(OUT_DIM)

NC, NS, L = 2, 16, 16
NW = NC * NS

K = 64


def _gather_sc(ids, e0, e1, e2):
    n = ids.shape[0]
    bw = n // NW
    mesh = plsc.VectorSubcoreMesh(core_axis_name="c", subcore_axis_name="s",
                                  num_cores=NC, num_subcores=NS)

    @functools.partial(
        pl.kernel,
        out_type=(
            jax.ShapeDtypeStruct((n, D0 // 2), jnp.int32),
            jax.ShapeDtypeStruct((n, D1 // 2), jnp.int32),
            jax.ShapeDtypeStruct((n, D2P // 2), jnp.int32),
        ),
        mesh=mesh,
        scratch_types=[
            pltpu.VMEM((bw,), jnp.int32),
            pltpu.VMEM((bw,), jnp.int32),
            pltpu.VMEM((bw,), jnp.int32),
            pltpu.VMEM((bw,), jnp.int32),
            pltpu.VMEM((2, K, D0 // 2), jnp.int32),
            pltpu.VMEM((2, K, D1 // 2), jnp.int32),
            pltpu.VMEM((2, K, D2P // 2), jnp.int32),
            pltpu.SemaphoreType.DMA,
            pltpu.SemaphoreType.DMA,
            pltpu.SemaphoreType.DMA,
            pltpu.SemaphoreType.DMA,
        ],
    )
    def gather_kernel(ids_hbm, e0_hbm, e1_hbm, e2_hbm,
                      g0_hbm, g1_hbm, g2_hbm,
                      ids_v, i0_v, i1_v, i2_v, r0_v, r1_v, r2_v,
                      sg0, sg1, sw0, sw1):
        sem_g = (sg0, sg1)
        sem_w = (sw0, sw1)
        wid = lax.axis_index("s") * NC + lax.axis_index("c")
        base = wid * bw
        pltpu.sync_copy(ids_hbm.at[pl.ds(base, bw)], ids_v)

        zero = jnp.zeros((L,), jnp.int32)

        def idx_body(i, _):
            ids_vec = ids_v[pl.ds(i * L, L)]
            i0_v[pl.ds(i * L, L)] = jnp.where(ids_vec < C0, ids_vec, zero)
            i1_v[pl.ds(i * L, L)] = jnp.where(
                (ids_vec >= C0) & (ids_vec < C1), ids_vec - C0, zero)
            i2_v[pl.ds(i * L, L)] = jnp.where(ids_vec >= C1, ids_vec - C1, zero)
            return 0

        lax.fori_loop(0, bw // L, idx_body, 0)

        def band(idx_v, e_hbm, g_hbm, r_v):
            nch = bw // K

            def gather(c, b):
                pltpu.async_copy(e_hbm.at[idx_v.at[pl.ds(c * K, K)]],
                                 r_v.at[b], sem_g[b])

            def wait_gather(b):
                pltpu.make_async_copy(e_hbm.at[idx_v.at[pl.ds(0, K)]],
                                      r_v.at[b], sem_g[b]).wait()

            def writeback(c, b):
                pltpu.async_copy(r_v.at[b], g_hbm.at[pl.ds(base + c * K, K)],
                                 sem_w[b])

            def wait_writeback(b):
                pltpu.make_async_copy(e_hbm.at[idx_v.at[pl.ds(0, K)]],
                                      r_v.at[b], sem_w[b]).wait()

            for b in range(2):
                gather(b, b)

            def body(i, _):
                for b in range(2):
                    c = i * 2 + b

                    @pl.when(c < nch)
                    def _():
                        wait_gather(b)
                        writeback(c, b)

                        @pl.when(c + 2 < nch)
                        def _():
                            wait_writeback(b)
                            gather(c + 2, b)
                return 0

            lax.fori_loop(0, (nch + 1) // 2, body, 0)

            @pl.when(nch >= 2)
            def _():
                wait_writeback(0)
                wait_writeback(1)

            @pl.when(nch == 1)
            def _():
                wait_writeback(0)

        band(i0_v, e0_hbm, g0_hbm, r0_v)
        band(i1_v, e1_hbm, g1_hbm, r1_v)
        band(i2_v, e2_hbm, g2_hbm, r2_v)

    return gather_kernel(ids, e0, e1, e2)


def _matmul_tc(g0, g1, g2, p0, p1, p2):
    n = g0.shape[0]
    bm = 512

    def mm_kernel(g0_ref, g1_ref, g2_ref, p0_ref, p1_ref, p2_ref, out_ref):
        dn = (((1,), (1,)), ((), ()))
        acc = lax.dot_general(g0_ref[...], p0_ref[...], dn,
                              preferred_element_type=jnp.float32)
        acc += lax.dot_general(g1_ref[...], p1_ref[...], dn,
                               preferred_element_type=jnp.float32)
        acc += lax.dot_general(g2_ref[...], p2_ref[...], dn,
                               preferred_element_type=jnp.float32)
        out_ref[...] = SCALE * acc

    return pl.pallas_call(
        mm_kernel,
        grid=(n // bm,),
        in_specs=[
            pl.BlockSpec((bm, D0), lambda i: (i, 0)),
            pl.BlockSpec((bm, D1), lambda i: (i, 0)),
            pl.BlockSpec((bm, D2P), lambda i: (i, 0)),
            pl.BlockSpec((OUT_DIM, D0), lambda i: (0, 0)),
            pl.BlockSpec((OUT_DIM, D1), lambda i: (0, 0)),
            pl.BlockSpec((OUT_DIM, D2P), lambda i: (0, 0)),
        ],
        out_specs=pl.BlockSpec((bm, OUT_DIM), lambda i: (i, 0)),
        out_shape=jax.ShapeDtypeStruct((n, OUT_DIM), jnp.float32),
    )(g0, g1, g2, p0, p1, p2)


def kernel(input_ids, embed0, proj0, embed1, proj1, embed2, proj2):
    b, s = input_ids.shape
    ids = input_ids.reshape(-1)
    def pack(x):
        v, d = x.shape
        return lax.bitcast_convert_type(
            x.astype(jnp.bfloat16).reshape(v, d // 2, 2), jnp.int32)

    def unpack(x):
        nn, d = x.shape
        return lax.bitcast_convert_type(x, jnp.bfloat16).reshape(nn, 2 * d)

    e2p = jnp.pad(embed2, ((0, 0), (0, D2P - D2)))
    p0b = proj0.astype(jnp.bfloat16)
    p1b = proj1.astype(jnp.bfloat16)
    p2b = jnp.pad(proj2, ((0, 0), (0, D2P - D2))).astype(jnp.bfloat16)
    g0, g1, g2 = _gather_sc(ids, pack(embed0), pack(embed1), pack(e2p))
    out = _matmul_tc(unpack(g0), unpack(g1), unpack(g2), p0b, p1b, p2b)
    return out.reshape(b, s, OUT_DIM)


# speedup vs baseline: 1.4724x; 1.4724x over previous
import functools
import math

import jax
import jax.numpy as jnp
from jax import lax
from jax.experimental import pallas as pl
from jax.experimental.pallas import tpu as pltpu
from jax.experimental.pallas import tpu_sc as plsc

C0, C1 = 20000, 60000
D0, D1, D2 = 1024, 256, 64
D2P = 256
OUT_DIM = 1024
SCALE = math.sqrt(OUT_DIM)

NC, NS, L = 2, 16, 16
NW = NC * NS

K = 32


def _gather_sc(ids, pos, e0, e1, e2):
    n = ids.shape[0]
    bw = n // NW
    mesh = plsc.VectorSubcoreMesh(core_axis_name="c", subcore_axis_name="s",
                                  num_cores=NC, num_subcores=NS)

    @functools.partial(
        pl.kernel,
        out_type=(
            jax.ShapeDtypeStruct((n, D0), jnp.float32),
            jax.ShapeDtypeStruct((n, D1), jnp.float32),
            jax.ShapeDtypeStruct((n, D2P), jnp.float32),
        ),
        mesh=mesh,
        scratch_types=[
            pltpu.VMEM((bw,), jnp.int32),
            pltpu.VMEM((bw,), jnp.int32),
            pltpu.VMEM((bw,), jnp.int32),
            pltpu.VMEM((bw,), jnp.int32),
            pltpu.VMEM((bw,), jnp.int32),
            pltpu.VMEM((2, K), jnp.int32),
            pltpu.VMEM((2, K, D0), jnp.float32),
            pltpu.VMEM((2, K, D1), jnp.float32),
            pltpu.VMEM((2, K, D2P), jnp.float32),
            pltpu.SemaphoreType.DMA,
            pltpu.SemaphoreType.DMA,
            pltpu.SemaphoreType.DMA,
            pltpu.SemaphoreType.DMA,
        ],
    )
    def gather_kernel(ids_hbm, pos_hbm, e0_hbm, e1_hbm, e2_hbm,
                      g0_hbm, g1_hbm, g2_hbm,
                      ids_v, i0_v, i1_v, i2_v, pos_v_all, posc_v,
                      r0_v, r1_v, r2_v,
                      sg0, sg1, sw0, sw1):
        sem_g = (sg0, sg1)
        sem_w = (sw0, sw1)
        wid = lax.axis_index("s") * NC + lax.axis_index("c")
        base = wid * bw
        pltpu.sync_copy(ids_hbm.at[pl.ds(base, bw)], ids_v)
        pltpu.sync_copy(pos_hbm.at[pl.ds(base, bw)], pos_v_all)

        zero = jnp.zeros((L,), jnp.int32)

        def idx_body(i, _):
            ids_vec = ids_v[pl.ds(i * L, L)]
            i0_v[pl.ds(i * L, L)] = jnp.where(ids_vec < C0, ids_vec, zero)
            i1_v[pl.ds(i * L, L)] = jnp.where(
                (ids_vec >= C0) & (ids_vec < C1), ids_vec - C0, zero)
            i2_v[pl.ds(i * L, L)] = jnp.where(ids_vec >= C1, ids_vec - C1, zero)
            return 0

        lax.fori_loop(0, bw // L, idx_body, 0)

        def band(idx_v, e_hbm, g_hbm, r_v):
            nch = bw // K

            def gather(c, b):
                pltpu.async_copy(e_hbm.at[idx_v.at[pl.ds(c * K, K)]],
                                 r_v.at[b], sem_g[b])

            def wait_gather(b):
                pltpu.make_async_copy(e_hbm.at[idx_v.at[pl.ds(0, K)]],
                                      r_v.at[b], sem_g[b]).wait()

            def writeback(c, b):
                for j in range(K // L):
                    posc_v[b, pl.ds(j * L, L)] = pos_v_all[
                        pl.ds(c * K + j * L, L)]
                pltpu.async_copy(r_v.at[b], g_hbm.at[posc_v.at[b]],
                                 sem_w[b])

            def wait_writeback(b):
                pltpu.make_async_copy(e_hbm.at[idx_v.at[pl.ds(0, K)]],
                                      r_v.at[b], sem_w[b]).wait()

            for b in range(2):
                gather(b, b)

            def body(i, _):
                for b in range(2):
                    c = i * 2 + b

                    @pl.when(c < nch)
                    def _():
                        wait_gather(b)
                        writeback(c, b)

                        @pl.when(c + 2 < nch)
                        def _():
                            wait_writeback(b)
                            gather(c + 2, b)
                return 0

            lax.fori_loop(0, (nch + 1) // 2, body, 0)

            @pl.when(nch >= 2)
            def _():
                wait_writeback(0)
                wait_writeback(1)

            @pl.when(nch == 1)
            def _():
                wait_writeback(0)

        band(i0_v, e0_hbm, g0_hbm, r0_v)
        band(i1_v, e1_hbm, g1_hbm, r1_v)
        band(i2_v, e2_hbm, g2_hbm, r2_v)

    return gather_kernel(ids, pos, e0, e1, e2)


def _matmul_tc(g0, g1, g2, p0, p1, p2):
    n = g0.shape[0]
    bm = 512

    def mm_kernel(g0_ref, g1_ref, g2_ref, p0_ref, p1_ref, p2_ref, out_ref):
        dn = (((1,), (1,)), ((), ()))
        acc = lax.dot_general(g0_ref[...], p0_ref[...], dn,
                              preferred_element_type=jnp.float32)
        acc += lax.dot_general(g1_ref[...], p1_ref[...], dn,
                               preferred_element_type=jnp.float32)
        acc += lax.dot_general(g2_ref[...], p2_ref[...], dn,
                               preferred_element_type=jnp.float32)
        out_ref[...] = SCALE * acc

    return pl.pallas_call(
        mm_kernel,
        grid=(n // bm,),
        in_specs=[
            pl.BlockSpec((bm, D0), lambda i: (i, 0)),
            pl.BlockSpec((bm, D1), lambda i: (i, 0)),
            pl.BlockSpec((bm, D2P), lambda i: (i, 0)),
            pl.BlockSpec((OUT_DIM, D0), lambda i: (0, 0)),
            pl.BlockSpec((OUT_DIM, D1), lambda i: (0, 0)),
            pl.BlockSpec((OUT_DIM, D2P), lambda i: (0, 0)),
        ],
        out_specs=pl.BlockSpec((bm, OUT_DIM), lambda i: (i, 0)),
        out_shape=jax.ShapeDtypeStruct((n, OUT_DIM), jnp.float32),
    )(g0, g1, g2, p0, p1, p2)


def kernel(input_ids, embed0, proj0, embed1, proj1, embed2, proj2):
    b, s = input_ids.shape
    ids = input_ids.reshape(-1)
    e2p = jnp.pad(embed2, ((0, 0), (0, D2P - D2)))
    p2pad = jnp.pad(proj2, ((0, 0), (0, D2P - D2)))
    pos = jnp.arange(ids.shape[0], dtype=jnp.int32)
    g0, g1, g2 = _gather_sc(ids, pos, embed0, embed1, e2p)
    out = _matmul_tc(g0, g1, g2, proj0, proj1, p2pad)
    return out.reshape(b, s, OUT_DIM)


# speedup vs baseline: 2.1039x; 1.4289x over previous
import functools
import math

import jax
import jax.numpy as jnp
from jax import lax
from jax.experimental import pallas as pl
from jax.experimental.pallas import tpu as pltpu
from jax.experimental.pallas import tpu_sc as plsc

C0, C1 = 20000, 60000
D0, D1, D2 = 1024, 256, 64
D2P = 256
OUT_DIM = 1024
SCALE = math.sqrt(OUT_DIM)

NC, NS, L = 2, 16, 16
NW = NC * NS

K = 32
CAP0, CAP1, CAP2 = 448, 768, 768


def _compact_tc(ids3, n):
    nw, tb, eight = ids3.shape

    def ck(ids_ref, i0_ref, p0_ref, i1_ref, p1_ref, i2_ref, p2_ref, cnt_ref):
        w = pl.program_id(0)
        ids = ids_ref[0]
        idsf = ids.astype(jnp.float32)
        rowi = lax.broadcasted_iota(jnp.int32, (tb, eight), 0).astype(
            jnp.float32)
        coli = lax.broadcasted_iota(jnp.int32, (tb, eight), 1).astype(
            jnp.float32)
        posf = w * (tb * eight) + rowi * eight + coli
        tril = jnp.where(
            lax.broadcasted_iota(jnp.int32, (tb, tb), 0)
            >= lax.broadcasted_iota(jnp.int32, (tb, tb), 1), 1.0, 0.0)
        su8 = jnp.where(
            lax.broadcasted_iota(jnp.int32, (eight, eight), 0)
            < lax.broadcasted_iota(jnp.int32, (eight, eight), 1), 1.0, 0.0)
        dnc = (((1,), (0,)), ((), ()))
        dn0 = (((0,), (0,)), ((), ()))

        def band(m, local, cap, idx_ref, pos_ref):
            mf = jnp.where(m, 1.0, 0.0)
            pre = lax.dot_general(tril, mf, dnc,
                                  preferred_element_type=jnp.float32)
            t = pre[tb - 1:tb, :]
            offs = lax.dot_general(t, su8, dnc,
                                   preferred_element_type=jnp.float32)
            dest = jnp.where(m, pre - 1.0 + offs, -1.0)
            iq = lax.broadcasted_iota(jnp.int32, (1, cap), 1).astype(
                jnp.float32)
            acc_i = jnp.zeros((1, cap), jnp.float32)
            acc_p = jnp.zeros((1, cap), jnp.float32)
            for j in range(eight):
                oh = jnp.where(dest[:, j:j + 1] == iq, 1.0, 0.0)
                acc_i += lax.dot_general(local[:, j:j + 1], oh, dn0,
                                         preferred_element_type=jnp.float32)
                acc_p += lax.dot_general(posf[:, j:j + 1] - n, oh, dn0,
                                         preferred_element_type=jnp.float32)
            idx_ref[0] = acc_i.astype(jnp.int32)
            pos_ref[0] = (acc_p + n).astype(jnp.int32)
            return offs + t

        c0 = band(ids < C0, idsf, CAP0, i0_ref, p0_ref)
        c1 = band((ids >= C0) & (ids < C1), idsf - C0, CAP1, i1_ref, p1_ref)
        c2 = band(ids >= C1, idsf - C1, CAP2, i2_ref, p2_ref)
        cnt_ref[0] = jnp.concatenate([c0, c1, c2, c0],
                                     axis=1).astype(jnp.int32)

    return pl.pallas_call(
        ck,
        grid=(nw,),
        in_specs=[pl.BlockSpec((1, tb, eight), lambda i: (i, 0, 0))],
        out_specs=[
            pl.BlockSpec((1, 1, CAP0), lambda i: (i, 0, 0)),
            pl.BlockSpec((1, 1, CAP0), lambda i: (i, 0, 0)),
            pl.BlockSpec((1, 1, CAP1), lambda i: (i, 0, 0)),
            pl.BlockSpec((1, 1, CAP1), lambda i: (i, 0, 0)),
            pl.BlockSpec((1, 1, CAP2), lambda i: (i, 0, 0)),
            pl.BlockSpec((1, 1, CAP2), lambda i: (i, 0, 0)),
            pl.BlockSpec((1, 1, 32), lambda i: (i, 0, 0)),
        ],
        out_shape=[
            jax.ShapeDtypeStruct((nw, 1, CAP0), jnp.int32),
            jax.ShapeDtypeStruct((nw, 1, CAP0), jnp.int32),
            jax.ShapeDtypeStruct((nw, 1, CAP1), jnp.int32),
            jax.ShapeDtypeStruct((nw, 1, CAP1), jnp.int32),
            jax.ShapeDtypeStruct((nw, 1, CAP2), jnp.int32),
            jax.ShapeDtypeStruct((nw, 1, CAP2), jnp.int32),
            jax.ShapeDtypeStruct((nw, 1, 32), jnp.int32),
        ],
    )(ids3)


def _make_sc_kernel(n, fast):
    bw = n // NW
    mesh = plsc.VectorSubcoreMesh(core_axis_name="c", subcore_axis_name="s",
                                  num_cores=NC, num_subcores=NS)

    @functools.partial(
        pl.kernel,
        out_type=(
            jax.ShapeDtypeStruct((n + 8, D0), jnp.float32),
            jax.ShapeDtypeStruct((n + 8, D1), jnp.float32),
            jax.ShapeDtypeStruct((n + 8, D2P), jnp.float32),
        ),
        mesh=mesh,
        scratch_types=[
            pltpu.VMEM((bw,), jnp.int32),
            pltpu.VMEM((CAP0,), jnp.int32),
            pltpu.VMEM((CAP0,), jnp.int32),
            pltpu.VMEM((CAP1,), jnp.int32),
            pltpu.VMEM((CAP1,), jnp.int32),
            pltpu.VMEM((CAP2,), jnp.int32),
            pltpu.VMEM((CAP2,), jnp.int32),
            pltpu.VMEM((bw,), jnp.int32),
            pltpu.VMEM((2 * K,), jnp.int32),
            pltpu.VMEM((2, K), jnp.int32),
            pltpu.VMEM((2, K, D0), jnp.float32),
            pltpu.VMEM((2, K, D1), jnp.float32),
            pltpu.VMEM((2, K, D2P), jnp.float32),
            pltpu.SemaphoreType.DMA,
            pltpu.SemaphoreType.DMA,
            pltpu.SemaphoreType.DMA,
            pltpu.SemaphoreType.DMA,
        ],
    )
    def sc_kernel(ids_hbm, pid_hbm, i0_hbm, p0_hbm, i1_hbm, p1_hbm,
                  i2_hbm, p2_hbm, e0_hbm, e1_hbm, e2_hbm,
                  g0_hbm, g1_hbm, g2_hbm,
                  ids_v, i0_v, p0_v, i1_v, p1_v, i2_v, p2_v, pid_v,
                  idxd_v, posc_v, r0_v, r1_v, r2_v, sg0, sg1, sw0, sw1):
        sem_g = (sg0, sg1)
        sem_w = (sw0, sw1)
        wid = lax.axis_index("s") * NC + lax.axis_index("c")
        base = wid * bw

        if fast:
            pltpu.sync_copy(i0_hbm.at[pl.ds(wid * CAP0, CAP0)], i0_v)
            pltpu.sync_copy(p0_hbm.at[pl.ds(wid * CAP0, CAP0)], p0_v)
            pltpu.sync_copy(i1_hbm.at[pl.ds(wid * CAP1, CAP1)], i1_v)
            pltpu.sync_copy(p1_hbm.at[pl.ds(wid * CAP1, CAP1)], p1_v)
            pltpu.sync_copy(i2_hbm.at[pl.ds(wid * CAP2, CAP2)], i2_v)
            pltpu.sync_copy(p2_hbm.at[pl.ds(wid * CAP2, CAP2)], p2_v)
        else:
            pltpu.sync_copy(ids_hbm.at[pl.ds(base, bw)], ids_v)
            pltpu.sync_copy(pid_hbm.at[pl.ds(base, bw)], pid_v)

        def run_band(e_hbm, g_hbm, r_v, nch, stage, idx_ref):
            def gather(c, b):
                stage(c, b)
                pltpu.async_copy(e_hbm.at[idx_ref(c, b)], r_v.at[b],
                                 sem_g[b])

            def wait_gather(b):
                pltpu.make_async_copy(e_hbm.at[idx_ref(0, b)], r_v.at[b],
                                      sem_g[b]).wait()

            def scatter(b):
                pltpu.async_copy(r_v.at[b], g_hbm.at[posc_v.at[b]], sem_w[b])

            def wait_scatter(b):
                pltpu.make_async_copy(e_hbm.at[idx_ref(0, b)], r_v.at[b],
                                      sem_w[b]).wait()

            for b in range(2):
                gather(b, b)

            def body(i, _):
                for b in range(2):
                    c = i * 2 + b

                    @pl.when(c < nch)
                    def _():
                        wait_gather(b)
                        scatter(b)

                        @pl.when(c + 2 < nch)
                        def _():
                            wait_scatter(b)
                            gather(c + 2, b)
                return 0

            lax.fori_loop(0, (nch + 1) // 2, body, 0)

            @pl.when(nch >= 2)
            def _():
                wait_scatter(0)
                wait_scatter(1)

            @pl.when(nch == 1)
            def _():
                wait_scatter(0)

        if fast:
            def make_stage(pos_v):
                def stage(c, b):
                    for j in range(K // L):
                        posc_v[b, pl.ds(j * L, L)] = pos_v[
                            pl.ds(c * K + j * L, L)]
                return stage

            run_band(e0_hbm, g0_hbm, r0_v, CAP0 // K, make_stage(p0_v),
                     lambda c, b: i0_v.at[pl.ds(c * K, K)])
            run_band(e1_hbm, g1_hbm, r1_v, CAP1 // K, make_stage(p1_v),
                     lambda c, b: i1_v.at[pl.ds(c * K, K)])
            run_band(e2_hbm, g2_hbm, r2_v, CAP2 // K, make_stage(p2_v),
                     lambda c, b: i2_v.at[pl.ds(c * K, K)])
        else:
            zero = jnp.zeros((L,), jnp.int32)

            def make_stage(sel):
                def stage(c, b):
                    for j in range(K // L):
                        v = ids_v[pl.ds(c * K + j * L, L)]
                        idxd_v[pl.ds(b * K + j * L, L)] = sel(v)
                        posc_v[b, pl.ds(j * L, L)] = pid_v[
                            pl.ds(c * K + j * L, L)]
                return stage

            didx = lambda c, b: idxd_v.at[pl.ds(b * K, K)]
            run_band(e0_hbm, g0_hbm, r0_v, bw // K,
                     make_stage(lambda v: jnp.where(v < C0, v, zero)), didx)
            run_band(e1_hbm, g1_hbm, r1_v, bw // K,
                     make_stage(lambda v: jnp.where(
                         (v >= C0) & (v < C1), v - C0, zero)), didx)
            run_band(e2_hbm, g2_hbm, r2_v, bw // K,
                     make_stage(lambda v: jnp.where(v >= C1, v - C1, zero)),
                     didx)

    return sc_kernel


def _matmul_tc(ids_col, g0, g1, g2, p0, p1, p2):
    n = ids_col.shape[0]
    bm = 512

    def mm_kernel(ids_ref, g0_ref, g1_ref, g2_ref,
                  p0_ref, p1_ref, p2_ref, out_ref):
        dn = (((1,), (1,)), ((), ()))
        ids_blk = ids_ref[...]
        a0 = lax.dot_general(g0_ref[...], p0_ref[...], dn,
                             preferred_element_type=jnp.float32)
        a1 = lax.dot_general(g1_ref[...], p1_ref[...], dn,
                             preferred_element_type=jnp.float32)
        a2 = lax.dot_general(g2_ref[...], p2_ref[...], dn,
                             preferred_element_type=jnp.float32)
        m0 = ids_blk < C0
        m1 = (ids_blk >= C0) & (ids_blk < C1)
        out_ref[...] = SCALE * jnp.where(m0, a0, jnp.where(m1, a1, a2))

    return pl.pallas_call(
        mm_kernel,
        grid=(n // bm,),
        in_specs=[
            pl.BlockSpec((bm, 1), lambda i: (i, 0)),
            pl.BlockSpec((bm, D0), lambda i: (i, 0)),
            pl.BlockSpec((bm, D1), lambda i: (i, 0)),
            pl.BlockSpec((bm, D2P), lambda i: (i, 0)),
            pl.BlockSpec((OUT_DIM, D0), lambda i: (0, 0)),
            pl.BlockSpec((OUT_DIM, D1), lambda i: (0, 0)),
            pl.BlockSpec((OUT_DIM, D2P), lambda i: (0, 0)),
        ],
        out_specs=pl.BlockSpec((bm, OUT_DIM), lambda i: (i, 0)),
        out_shape=jax.ShapeDtypeStruct((n, OUT_DIM), jnp.float32),
    )(ids_col, g0, g1, g2, p0, p1, p2)


def kernel(input_ids, embed0, proj0, embed1, proj1, embed2, proj2):
    b, s = input_ids.shape
    ids = input_ids.reshape(-1)
    n = ids.shape[0]
    bw = n // NW
    e2p = jnp.pad(embed2, ((0, 0), (0, D2P - D2)))
    p2p = jnp.pad(proj2, ((0, 0), (0, D2P - D2)))

    i0, po0, i1, po1, i2, po2, cnt = _compact_tc(
        ids.reshape(NW, bw // 8, 8), n)

    idsw = ids.reshape(NW, bw)
    wk = jnp.broadcast_to(jnp.arange(NW)[:, None], (NW, bw))
    posw = jnp.broadcast_to(jnp.arange(n).reshape(NW, bw), (NW, bw))

    def mklists(m, local, cap):
        d = jnp.cumsum(m, axis=1) - 1
        col = jnp.where(m, d, cap)
        idx = jnp.zeros((NW, cap + 1), jnp.int32).at[wk, col].set(
            jnp.where(m, local, 0), mode='drop')[:, :cap]
        pos = jnp.full((NW, cap + 1), n, jnp.int32).at[wk, col].set(
            jnp.where(m, posw, n), mode='drop')[:, :cap]
        return idx.reshape(NW, 1, cap), pos.reshape(NW, 1, cap), m.sum(axis=1)

    m0w = idsw < C0
    m1w = (idsw >= C0) & (idsw < C1)
    m2w = idsw >= C1
    i0, po0, cnt0 = mklists(m0w, idsw, CAP0)
    i1, po1, cnt1 = mklists(m1w, idsw - C0, CAP1)
    i2, po2, cnt2 = mklists(m2w, idsw - C1, CAP2)
    ok = ((jnp.max(cnt0) <= CAP0) & (jnp.max(cnt1) <= CAP1)
          & (jnp.max(cnt2) <= CAP2))

    pid = jnp.arange(n, dtype=jnp.int32)
    fast_k = _make_sc_kernel(n, fast=True)
    dense_k = _make_sc_kernel(n, fast=False)
    flat = lambda x: x.reshape(-1)
    args = (ids, pid, flat(i0), flat(po0), flat(i1), flat(po1), flat(i2),
            flat(po2), embed0, embed1, e2p)
    g0, g1, g2 = lax.cond(ok, lambda a: fast_k(*a), lambda a: dense_k(*a),
                          args)

    out = _matmul_tc(ids.reshape(n, 1), g0, g1, g2, proj0, proj1, p2p)
    return out.reshape(b, s, OUT_DIM)


# speedup vs baseline: 4.5134x; 2.1452x over previous
import functools
import math

import jax
import jax.numpy as jnp
from jax import lax
from jax.experimental import pallas as pl
from jax.experimental.pallas import tpu as pltpu
from jax.experimental.pallas import tpu_sc as plsc

C0, C1 = 20000, 60000
D0, D1, D2 = 1024, 256, 64
D2P = 256
OUT_DIM = 1024
SCALE = math.sqrt(OUT_DIM)

NC, NS, L = 2, 16, 16
NW = NC * NS

K = 32
CAP0, CAP1, CAP2 = 448, 768, 768


def _compact_tc(ids3, n):
    nw, tb, eight = ids3.shape

    def ck(ids_ref, i0_ref, p0_ref, i1_ref, p1_ref, i2_ref, p2_ref, cnt_ref):
        w = pl.program_id(0)
        ids = ids_ref[0]
        idsf = ids.astype(jnp.float32)
        rowi = lax.broadcasted_iota(jnp.int32, (tb, eight), 0).astype(
            jnp.float32)
        coli = lax.broadcasted_iota(jnp.int32, (tb, eight), 1).astype(
            jnp.float32)
        posf = w * (tb * eight) + rowi * eight + coli
        tril = jnp.where(
            lax.broadcasted_iota(jnp.int32, (tb, tb), 0)
            >= lax.broadcasted_iota(jnp.int32, (tb, tb), 1), 1.0, 0.0)
        su8 = jnp.where(
            lax.broadcasted_iota(jnp.int32, (eight, eight), 0)
            < lax.broadcasted_iota(jnp.int32, (eight, eight), 1), 1.0, 0.0)
        dnc = (((1,), (0,)), ((), ()))
        dn0 = (((0,), (0,)), ((), ()))

        def band(m, local, cap, idx_ref, pos_ref):
            mf = jnp.where(m, 1.0, 0.0)
            pre = lax.dot_general(tril, mf, dnc,
                                  preferred_element_type=jnp.float32)
            t = pre[tb - 1:tb, :]
            offs = lax.dot_general(t, su8, dnc,
                                   precision=lax.Precision.HIGHEST,
                                   preferred_element_type=jnp.float32)
            dest = jnp.where(m, pre - 1.0 + offs, -1.0)
            iq = lax.broadcasted_iota(jnp.int32, (1, cap), 1).astype(
                jnp.float32)
            acc_i = jnp.zeros((1, cap), jnp.float32)
            acc_p = jnp.zeros((1, cap), jnp.float32)
            for j in range(eight):
                oh = jnp.where(dest[:, j:j + 1] == iq, 1.0, 0.0)
                lj = jnp.transpose(local[:, j:j + 1])
                pj = jnp.transpose(posf[:, j:j + 1] - n)
                acc_i += lax.dot_general(lj, oh, dnc,
                                         precision=lax.Precision.HIGHEST,
                                         preferred_element_type=jnp.float32)
                acc_p += lax.dot_general(pj, oh, dnc,
                                         precision=lax.Precision.HIGHEST,
                                         preferred_element_type=jnp.float32)
            idx_ref[0] = acc_i.astype(jnp.int32)
            pos_ref[0] = (acc_p + n).astype(jnp.int32)
            return offs + t

        c0 = band(ids < C0, idsf, CAP0, i0_ref, p0_ref)
        c1 = band((ids >= C0) & (ids < C1), idsf - C0, CAP1, i1_ref, p1_ref)
        c2 = band(ids >= C1, idsf - C1, CAP2, i2_ref, p2_ref)
        cnt_ref[0] = jnp.concatenate([c0, c1, c2, c0],
                                     axis=1).astype(jnp.int32)

    return pl.pallas_call(
        ck,
        grid=(nw,),
        in_specs=[pl.BlockSpec((1, tb, eight), lambda i: (i, 0, 0))],
        out_specs=[
            pl.BlockSpec((1, 1, CAP0), lambda i: (i, 0, 0)),
            pl.BlockSpec((1, 1, CAP0), lambda i: (i, 0, 0)),
            pl.BlockSpec((1, 1, CAP1), lambda i: (i, 0, 0)),
            pl.BlockSpec((1, 1, CAP1), lambda i: (i, 0, 0)),
            pl.BlockSpec((1, 1, CAP2), lambda i: (i, 0, 0)),
            pl.BlockSpec((1, 1, CAP2), lambda i: (i, 0, 0)),
            pl.BlockSpec((1, 1, 32), lambda i: (i, 0, 0)),
        ],
        out_shape=[
            jax.ShapeDtypeStruct((nw, 1, CAP0), jnp.int32),
            jax.ShapeDtypeStruct((nw, 1, CAP0), jnp.int32),
            jax.ShapeDtypeStruct((nw, 1, CAP1), jnp.int32),
            jax.ShapeDtypeStruct((nw, 1, CAP1), jnp.int32),
            jax.ShapeDtypeStruct((nw, 1, CAP2), jnp.int32),
            jax.ShapeDtypeStruct((nw, 1, CAP2), jnp.int32),
            jax.ShapeDtypeStruct((nw, 1, 32), jnp.int32),
        ],
    )(ids3)


def _make_sc_kernel(n, fast):
    bw = n // NW
    mesh = plsc.VectorSubcoreMesh(core_axis_name="c", subcore_axis_name="s",
                                  num_cores=NC, num_subcores=NS)

    @functools.partial(
        pl.kernel,
        out_type=(
            jax.ShapeDtypeStruct((n + 8, D0), jnp.float32),
            jax.ShapeDtypeStruct((n + 8, D1), jnp.float32),
            jax.ShapeDtypeStruct((n + 8, D2P), jnp.float32),
        ),
        mesh=mesh,
        scratch_types=[
            pltpu.VMEM((bw,), jnp.int32),
            pltpu.VMEM((CAP0,), jnp.int32),
            pltpu.VMEM((CAP0,), jnp.int32),
            pltpu.VMEM((CAP1,), jnp.int32),
            pltpu.VMEM((CAP1,), jnp.int32),
            pltpu.VMEM((CAP2,), jnp.int32),
            pltpu.VMEM((CAP2,), jnp.int32),
            pltpu.VMEM((bw,), jnp.int32),
            pltpu.VMEM((2 * K,), jnp.int32),
            pltpu.VMEM((2, K), jnp.int32),
            pltpu.VMEM((2, K, D0), jnp.float32),
            pltpu.VMEM((2, K, D1), jnp.float32),
            pltpu.VMEM((2, K, D2P), jnp.float32),
            pltpu.SemaphoreType.DMA,
            pltpu.SemaphoreType.DMA,
            pltpu.SemaphoreType.DMA,
            pltpu.SemaphoreType.DMA,
        ],
    )
    def sc_kernel(ids_hbm, pid_hbm, i0_hbm, p0_hbm, i1_hbm, p1_hbm,
                  i2_hbm, p2_hbm, e0_hbm, e1_hbm, e2_hbm,
                  g0_hbm, g1_hbm, g2_hbm,
                  ids_v, i0_v, p0_v, i1_v, p1_v, i2_v, p2_v, pid_v,
                  idxd_v, posc_v, r0_v, r1_v, r2_v, sg0, sg1, sw0, sw1):
        sem_g = (sg0, sg1)
        sem_w = (sw0, sw1)
        wid = lax.axis_index("s") * NC + lax.axis_index("c")
        base = wid * bw

        if fast:
            pltpu.sync_copy(i0_hbm.at[pl.ds(wid * CAP0, CAP0)], i0_v)
            pltpu.sync_copy(p0_hbm.at[pl.ds(wid * CAP0, CAP0)], p0_v)
            pltpu.sync_copy(i1_hbm.at[pl.ds(wid * CAP1, CAP1)], i1_v)
            pltpu.sync_copy(p1_hbm.at[pl.ds(wid * CAP1, CAP1)], p1_v)
            pltpu.sync_copy(i2_hbm.at[pl.ds(wid * CAP2, CAP2)], i2_v)
            pltpu.sync_copy(p2_hbm.at[pl.ds(wid * CAP2, CAP2)], p2_v)
        else:
            pltpu.sync_copy(ids_hbm.at[pl.ds(base, bw)], ids_v)
            pltpu.sync_copy(pid_hbm.at[pl.ds(base, bw)], pid_v)

        def run_band(e_hbm, g_hbm, r_v, nch, stage, idx_ref):
            def gather(c, b):
                stage(c, b)
                pltpu.async_copy(e_hbm.at[idx_ref(c, b)], r_v.at[b],
                                 sem_g[b])

            def wait_gather(b):
                pltpu.make_async_copy(e_hbm.at[idx_ref(0, b)], r_v.at[b],
                                      sem_g[b]).wait()

            def scatter(b):
                pltpu.async_copy(r_v.at[b], g_hbm.at[posc_v.at[b]], sem_w[b])

            def wait_scatter(b):
                pltpu.make_async_copy(e_hbm.at[idx_ref(0, b)], r_v.at[b],
                                      sem_w[b]).wait()

            for b in range(2):
                gather(b, b)

            def body(i, _):
                for b in range(2):
                    c = i * 2 + b

                    @pl.when(c < nch)
                    def _():
                        wait_gather(b)
                        scatter(b)

                        @pl.when(c + 2 < nch)
                        def _():
                            wait_scatter(b)
                            gather(c + 2, b)
                return 0

            lax.fori_loop(0, (nch + 1) // 2, body, 0)

            @pl.when(nch >= 2)
            def _():
                wait_scatter(0)
                wait_scatter(1)

            @pl.when(nch == 1)
            def _():
                wait_scatter(0)

        if fast:
            def make_stage(pos_v):
                def stage(c, b):
                    for j in range(K // L):
                        posc_v[b, pl.ds(j * L, L)] = pos_v[
                            pl.ds(c * K + j * L, L)]
                return stage

            run_band(e0_hbm, g0_hbm, r0_v, CAP0 // K, make_stage(p0_v),
                     lambda c, b: i0_v.at[pl.ds(c * K, K)])
            run_band(e1_hbm, g1_hbm, r1_v, CAP1 // K, make_stage(p1_v),
                     lambda c, b: i1_v.at[pl.ds(c * K, K)])
            run_band(e2_hbm, g2_hbm, r2_v, CAP2 // K, make_stage(p2_v),
                     lambda c, b: i2_v.at[pl.ds(c * K, K)])
        else:
            zero = jnp.zeros((L,), jnp.int32)

            def make_stage(sel):
                def stage(c, b):
                    for j in range(K // L):
                        v = ids_v[pl.ds(c * K + j * L, L)]
                        idxd_v[pl.ds(b * K + j * L, L)] = sel(v)
                        posc_v[b, pl.ds(j * L, L)] = pid_v[
                            pl.ds(c * K + j * L, L)]
                return stage

            didx = lambda c, b: idxd_v.at[pl.ds(b * K, K)]
            run_band(e0_hbm, g0_hbm, r0_v, bw // K,
                     make_stage(lambda v: jnp.where(v < C0, v, zero)), didx)
            run_band(e1_hbm, g1_hbm, r1_v, bw // K,
                     make_stage(lambda v: jnp.where(
                         (v >= C0) & (v < C1), v - C0, zero)), didx)
            run_band(e2_hbm, g2_hbm, r2_v, bw // K,
                     make_stage(lambda v: jnp.where(v >= C1, v - C1, zero)),
                     didx)

    return sc_kernel


def _matmul_tc(ids_col, g0, g1, g2, p0, p1, p2):
    n = ids_col.shape[0]
    bm = 512

    def mm_kernel(ids_ref, g0_ref, g1_ref, g2_ref,
                  p0_ref, p1_ref, p2_ref, out_ref):
        dn = (((1,), (1,)), ((), ()))
        ids_blk = ids_ref[...]
        a0 = lax.dot_general(g0_ref[...], p0_ref[...], dn,
                             preferred_element_type=jnp.float32)
        a1 = lax.dot_general(g1_ref[...], p1_ref[...], dn,
                             preferred_element_type=jnp.float32)
        a2 = lax.dot_general(g2_ref[...], p2_ref[...], dn,
                             preferred_element_type=jnp.float32)
        m0 = ids_blk < C0
        m1 = (ids_blk >= C0) & (ids_blk < C1)
        out_ref[...] = SCALE * jnp.where(m0, a0, jnp.where(m1, a1, a2))

    return pl.pallas_call(
        mm_kernel,
        grid=(n // bm,),
        in_specs=[
            pl.BlockSpec((bm, 1), lambda i: (i, 0)),
            pl.BlockSpec((bm, D0), lambda i: (i, 0)),
            pl.BlockSpec((bm, D1), lambda i: (i, 0)),
            pl.BlockSpec((bm, D2P), lambda i: (i, 0)),
            pl.BlockSpec((OUT_DIM, D0), lambda i: (0, 0)),
            pl.BlockSpec((OUT_DIM, D1), lambda i: (0, 0)),
            pl.BlockSpec((OUT_DIM, D2P), lambda i: (0, 0)),
        ],
        out_specs=pl.BlockSpec((bm, OUT_DIM), lambda i: (i, 0)),
        out_shape=jax.ShapeDtypeStruct((n, OUT_DIM), jnp.float32),
    )(ids_col, g0, g1, g2, p0, p1, p2)


def kernel(input_ids, embed0, proj0, embed1, proj1, embed2, proj2):
    b, s = input_ids.shape
    ids = input_ids.reshape(-1)
    n = ids.shape[0]
    bw = n // NW
    e2p = jnp.pad(embed2, ((0, 0), (0, D2P - D2)))
    p2p = jnp.pad(proj2, ((0, 0), (0, D2P - D2)))

    i0, po0, i1, po1, i2, po2, cnt = _compact_tc(
        ids.reshape(NW, bw // 8, 8), n)
    ok = ((jnp.max(cnt[:, 0, 7]) <= CAP0)
          & (jnp.max(cnt[:, 0, 15]) <= CAP1)
          & (jnp.max(cnt[:, 0, 23]) <= CAP2))

    pid = jnp.arange(n, dtype=jnp.int32)
    fast_k = _make_sc_kernel(n, fast=True)
    dense_k = _make_sc_kernel(n, fast=False)
    flat = lambda x: x.reshape(-1)
    args = (ids, pid, flat(i0), flat(po0), flat(i1), flat(po1), flat(i2),
            flat(po2), embed0, embed1, e2p)
    g0, g1, g2 = lax.cond(ok, lambda a: fast_k(*a), lambda a: dense_k(*a),
                          args)

    out = _matmul_tc(ids.reshape(n, 1), g0, g1, g2, proj0, proj1, p2p)
    return out.reshape(b, s, OUT_DIM)
